# baseline (speedup 1.0000x reference)
import jax
import jax.numpy as jnp
from jax import lax
from jax.experimental import pallas as pl
from jax.experimental.pallas import tpu as pltpu

_TB = 1024


def _router_tc_body(x_ref, w_ref, out_ref, sel_ref, wgt_ref):
    x = x_ref[...]
    w = w_ref[...]
    logits = lax.dot_general(x, w, (((1,), (1,)), ((), ())),
                             preferred_element_type=jnp.float32)
    out_ref[...] = logits
    m = jnp.max(logits, axis=-1, keepdims=True)
    ii = lax.broadcasted_iota(jnp.int32, logits.shape, 1)
    sel = jnp.min(jnp.where(logits == m, ii, jnp.int32(logits.shape[-1])),
                  axis=-1)
    sel_ref[...] = sel
    wgt_ref[...] = jnp.ones_like(sel).astype(jnp.float32)


def kernel(hidden_states, W):
    B, S, H = hidden_states.shape
    E = W.shape[0]
    T = B * S
    x = hidden_states.reshape(T, H)
    grid = (T // _TB,)
    logits, sel, wgt = pl.pallas_call(
        _router_tc_body,
        grid=grid,
        in_specs=[pl.BlockSpec((_TB, H), lambda i: (i, 0)),
                  pl.BlockSpec((E, H), lambda i: (0, 0))],
        out_specs=[pl.BlockSpec((_TB, E), lambda i: (i, 0)),
                   pl.BlockSpec((_TB,), lambda i: (i,)),
                   pl.BlockSpec((_TB,), lambda i: (i,))],
        out_shape=[jax.ShapeDtypeStruct((T, E), jnp.float32),
                   jax.ShapeDtypeStruct((T,), jnp.int32),
                   jax.ShapeDtypeStruct((T,), jnp.float32)],
        compiler_params=pltpu.CompilerParams(
            dimension_semantics=("arbitrary",)),
    )(x, W)
    return (logits.reshape(B, S, E), sel.reshape(B, S),
            wgt.reshape(B, S))
